# baseline (speedup 1.0000x reference)
import functools

import jax
import jax.numpy as jnp
from jax import lax
from jax.experimental import pallas as pl
from jax.experimental.pallas import tpu as pltpu
from jax.experimental.pallas import tpu_sc as plsc

N = 10000
E = 160000
D = 256
H = 8
C = 32
HC = H * C
NUM_CLASSES = 64

NSUB = 16
NP = 10112
TRASH = N
E2 = E + N
E2P = 172032
CHUNK = E2P // NSUB
ROWS = NP // NSUB

BN = 1000


def _augment(h, als, half):
    pad = jnp.zeros((h.shape[0], 16 - H), jnp.float32)
    return (jnp.concatenate([h[:, :half], als, pad], axis=1),
            jnp.concatenate([h[:, half:], als, pad], axis=1))


def _tc_first_body(x_ref, w_ref, as_ref, ad_ref, g0_ref, g1_ref, do_ref):
    h = jnp.dot(x_ref[...], w_ref[...], preferred_element_type=jnp.float32)
    half = h.shape[1] // 2
    als = jnp.dot(h, as_ref[...], preferred_element_type=jnp.float32)
    g0_ref[...], g1_ref[...] = _augment(h, als, half)
    do_ref[...] = jnp.dot(h, ad_ref[...], preferred_element_type=jnp.float32)


def _tc_mid_body(z0_ref, z1_ref, r_ref, g_ref, b_ref, w_ref, as_ref, ad_ref,
                 y_ref, g0_ref, g1_ref, do_ref):
    halfp = z0_ref.shape[1] - 16
    z = jnp.concatenate([z0_ref[:, :halfp], z1_ref[:, :halfp]], axis=1) + r_ref[...]
    mu = jnp.mean(z, axis=-1, keepdims=True)
    var = jnp.mean((z - mu) ** 2, axis=-1, keepdims=True)
    y = (z - mu) / jnp.sqrt(var + 1e-5) * g_ref[...] + b_ref[...]
    y = jnp.where(y > 0, y, jnp.exp(jnp.minimum(y, 0.0)) - 1.0)
    y_ref[...] = y
    h = jnp.dot(y, w_ref[...], preferred_element_type=jnp.float32)
    half = h.shape[1] // 2
    als = jnp.dot(h, as_ref[...], preferred_element_type=jnp.float32)
    g0_ref[...], g1_ref[...] = _augment(h, als, half)
    do_ref[...] = jnp.dot(h, ad_ref[...], preferred_element_type=jnp.float32)


def _tc_first(x, w, a_s, a_d):
    k, m = w.shape
    wid = m // 2 + 16
    f32 = jnp.float32
    return pl.pallas_call(
        _tc_first_body,
        grid=(N // BN,),
        in_specs=[
            pl.BlockSpec((BN, k), lambda i: (i, 0)),
            pl.BlockSpec((k, m), lambda i: (0, 0)),
            pl.BlockSpec((m, H), lambda i: (0, 0)),
            pl.BlockSpec((m, H), lambda i: (0, 0)),
        ],
        out_specs=[
            pl.BlockSpec((BN, wid), lambda i: (i, 0)),
            pl.BlockSpec((BN, wid), lambda i: (i, 0)),
            pl.BlockSpec((BN, H), lambda i: (i, 0)),
        ],
        out_shape=[
            jax.ShapeDtypeStruct((N, wid), f32),
            jax.ShapeDtypeStruct((N, wid), f32),
            jax.ShapeDtypeStruct((N, H), f32),
        ],
    )(x, w, a_s, a_d)


def _tc_mid(z0, z1, r, ln_g, ln_b, w, a_s, a_d):
    k, m = w.shape
    widp = z0.shape[1]
    wid = m // 2 + 16
    f32 = jnp.float32
    return pl.pallas_call(
        _tc_mid_body,
        grid=(N // BN,),
        in_specs=[
            pl.BlockSpec((BN, widp), lambda i: (i, 0)),
            pl.BlockSpec((BN, widp), lambda i: (i, 0)),
            pl.BlockSpec((BN, k), lambda i: (i, 0)),
            pl.BlockSpec((1, k), lambda i: (0, 0)),
            pl.BlockSpec((1, k), lambda i: (0, 0)),
            pl.BlockSpec((k, m), lambda i: (0, 0)),
            pl.BlockSpec((m, H), lambda i: (0, 0)),
            pl.BlockSpec((m, H), lambda i: (0, 0)),
        ],
        out_specs=[
            pl.BlockSpec((BN, k), lambda i: (i, 0)),
            pl.BlockSpec((BN, wid), lambda i: (i, 0)),
            pl.BlockSpec((BN, wid), lambda i: (i, 0)),
            pl.BlockSpec((BN, H), lambda i: (i, 0)),
        ],
        out_shape=[
            jax.ShapeDtypeStruct((N, k), f32),
            jax.ShapeDtypeStruct((N, wid), f32),
            jax.ShapeDtypeStruct((N, wid), f32),
            jax.ShapeDtypeStruct((N, H), f32),
        ],
    )(z0, z1, r, ln_g.reshape(1, k), ln_b.reshape(1, k), w, a_s, a_d)


def _splat(v, idx):
    dnums = lax.GatherDimensionNumbers(
        offset_dims=(), collapsed_slice_dims=(0,), start_index_map=(0,))
    return lax.gather(v, idx[:, None], dnums, (1,),
                      mode=lax.GatherScatterMode.PROMISE_IN_BOUNDS)


@functools.cache
def _make_sc_layer(half, c_l, B, NS, interpret=False):
    nj = half // 16
    f32 = jnp.float32
    mesh = plsc.VectorSubcoreMesh(core_axis_name="c", subcore_axis_name="s",
                                  num_cores=2, num_subcores=NSUB)

    def _head_splat(p, c, j):
        if c_l == 32:
            idx = jnp.full((16,), j // 2, jnp.int32) + c * (half // 32)
        else:
            idx = jnp.zeros((16,), jnp.int32)
        return _splat(p, idx)

    WID = half + 16
    TAIL = half
    T = CHUNK // B

    def body(*refs):
        (sd2_h, adt_h, h0_h, h1_h, bias_h, o0_h, o1_h,
         acc, sdbuf, dsc, adb, g, bias_v) = refs[:13]
        sems = refs[13:]
        ISEM = sems[0:NS]
        BSEM = sems[NS:2 * NS]
        GSEM = sems[2 * NS:3 * NS]
        SSEM = sems[3 * NS:4 * NS]
        c = lax.axis_index("c")
        s = lax.axis_index("s")
        rbase = s * ROWS

        def idx_issue(t, k):
            row = s * T + t
            pltpu.async_copy(sd2_h.at[pl.ds(row, 1)], sdbuf.at[pl.ds(k, 1)], ISEM[k])

        def idx_wait(k):
            pltpu.make_async_copy(sd2_h.at[pl.ds(0, 1)], sdbuf.at[pl.ds(k, 1)], ISEM[k]).wait()

        def gathers_issue(k):
            def sh(q, _):
                dsc[k, pl.ds(16 * q, 16)] = sdbuf[k, pl.ds(B + 16 * q, 16)]
                return 0
            lax.fori_loop(0, B // 16, sh, 0, unroll=True)
            pltpu.async_copy(adt_h.at[dsc.at[k]], adb.at[k], BSEM[k])

            @pl.when(c == 0)
            def _():
                pltpu.async_copy(h0_h.at[sdbuf.at[k, pl.ds(0, B)]], g.at[k], GSEM[k])

            @pl.when(c == 1)
            def _():
                pltpu.async_copy(h1_h.at[sdbuf.at[k, pl.ds(0, B)]], g.at[k], GSEM[k])

        def gathers_wait(k):
            pltpu.make_async_copy(adt_h.at[dsc.at[k]], adb.at[k], BSEM[k]).wait()
            pltpu.make_async_copy(h0_h.at[sdbuf.at[k, pl.ds(0, B)]], g.at[k], GSEM[k]).wait()

        def scatter_issue(k):
            pltpu.async_copy(g.at[k], acc.at[dsc.at[k]], SSEM[k], add=True)

        def scatter_drain(k):
            pltpu.make_async_copy(g.at[k], acc.at[dsc.at[k]], SSEM[k]).wait()

        pltpu.sync_copy(bias_h.at[pl.ds(c, 1)], bias_v)

        def _fill(i, _):
            for j in range(WID // 16):
                g[0, i, pl.ds(16 * j, 16)] = jnp.zeros((16,), f32)
            return 0
        lax.fori_loop(0, B, _fill, 0)
        zchunks = [(q * B, B) for q in range(ROWS // B)]
        if ROWS % B:
            zchunks.append((ROWS - ROWS % B, ROWS % B))
        for off, sz in zchunks:
            pltpu.async_copy(g.at[0, pl.ds(0, sz)], acc.at[pl.ds(rbase + off, sz)], GSEM[0])
        for off, sz in zchunks:
            pltpu.make_async_copy(g.at[0, pl.ds(0, sz)], acc.at[pl.ds(rbase + off, sz)], GSEM[0]).wait()
        plsc.subcore_barrier()

        def compute(k):
            def me(i, _):
                e = g[k, i, pl.ds(TAIL, 16)] + adb[k, i]
                e = jnp.maximum(e, 0.2 * e)
                p = jnp.exp(e)
                g[k, i, pl.ds(TAIL, 16)] = p
                step = max(1, 32 // c_l)
                sps = {j: _head_splat(p, c, j) for j in range(0, nj, step)}
                for j in range(nj):
                    g[k, i, pl.ds(16 * j, 16)] = g[k, i, pl.ds(16 * j, 16)] * sps[(j // step) * step]
                return 0
            lax.fori_loop(0, B, me, 0, unroll=8)

        idx_issue(0, 0)
        idx_issue(1, 1)
        idx_wait(0)
        gathers_issue(0)

        def tstep(t3, _):
            for ks in range(NS):
                t = NS * t3 + ks
                k1 = (ks + 1) % NS

                if NS > 2:
                    @pl.when(t + 2 < T)
                    def _():
                        idx_issue(t + 2, (ks + 2) % NS)

                @pl.when(t + 1 < T)
                def _():
                    idx_wait(k1)

                    @pl.when(t + 1 >= NS)
                    def _():
                        scatter_drain(k1)
                    gathers_issue(k1)
                gathers_wait(ks)
                if NS == 2:
                    @pl.when(t + 2 < T)
                    def _():
                        idx_issue(t + 2, ks)
                compute(ks)
                scatter_issue(ks)
            return 0
        lax.fori_loop(0, T // NS, tstep, 0)
        for ks in range(NS):
            scatter_drain(ks)
        plsc.subcore_barrier()

        rounds = []
        off = 0
        while off < ROWS:
            sizes = []
            for k in range(NS):
                sz = min(B, ROWS - off - sum(sizes))
                if sz > 0:
                    sizes.append(sz)
            rounds.append((off, sizes))
            off += sum(sizes)

        for off, sizes in rounds:
            for k, sz in enumerate(sizes):
                r0 = rbase + off + k * B
                pltpu.async_copy(acc.at[pl.ds(r0, sz)], g.at[k, pl.ds(0, sz)], GSEM[k])
            for k, sz in enumerate(sizes):
                r0 = rbase + off + k * B
                pltpu.make_async_copy(acc.at[pl.ds(r0, sz)], g.at[k, pl.ds(0, sz)], GSEM[k]).wait()

                def ne(i, _, k=k):
                    d = g[k, i, pl.ds(TAIL, 16)] + 1e-16
                    for j in range(nj):
                        bv = bias_v[0, pl.ds(16 * j, 16)]
                        g[k, i, pl.ds(16 * j, 16)] = g[k, i, pl.ds(16 * j, 16)] / _head_splat(d, c, j) + bv
                    return 0
                lax.fori_loop(0, sz, ne, 0, unroll=2)

                @pl.when(c == 0)
                def _():
                    pltpu.async_copy(g.at[k, pl.ds(0, sz)], o0_h.at[pl.ds(r0, sz)], BSEM[k])

                @pl.when(c == 1)
                def _():
                    pltpu.async_copy(g.at[k, pl.ds(0, sz)], o1_h.at[pl.ds(r0, sz)], BSEM[k])
            for k, sz in enumerate(sizes):
                r0 = rbase + off + k * B

                @pl.when(c == 0)
                def _():
                    pltpu.make_async_copy(g.at[k, pl.ds(0, sz)], o0_h.at[pl.ds(r0, sz)], BSEM[k]).wait()

                @pl.when(c == 1)
                def _():
                    pltpu.make_async_copy(g.at[k, pl.ds(0, sz)], o1_h.at[pl.ds(r0, sz)], BSEM[k]).wait()

    return pl.kernel(
        body,
        out_type=(
            jax.ShapeDtypeStruct((NP, WID), f32),
            jax.ShapeDtypeStruct((NP, WID), f32),
        ),
        mesh=mesh,
        interpret=interpret,
        compiler_params=pltpu.CompilerParams(use_tc_tiling_on_sc=False),
        scratch_types=[
            pltpu.VMEM_SHARED((NP, WID), f32),
            pltpu.VMEM((NS, 2 * B), jnp.int32),
            pltpu.VMEM((NS, B), jnp.int32),
            pltpu.VMEM((NS, B, 16), f32),
            pltpu.VMEM((NS, B, WID), f32),
            pltpu.VMEM((1, half), f32),
        ] + [pltpu.SemaphoreType.DMA] * (4 * NS),
    )


def _attn_mats(a_src, a_dst, heads, c_out):
    eye = jnp.eye(heads, dtype=jnp.float32)
    a_s = (a_src[:, :, None] * eye[:, None, :]).reshape(heads * c_out, heads)
    a_d = (a_dst[:, :, None] * eye[:, None, :]).reshape(heads * c_out, heads)
    if heads < H:
        a_s = jnp.pad(a_s, ((0, 0), (0, H - heads)))
        a_d = jnp.pad(a_d, ((0, 0), (0, H - heads)))
    return a_s, a_d


def _pad_tab(a):
    return jnp.pad(a, ((0, NP - N), (0, 16 - H)))


def _sc_edge_stage(fn, sd2, ald, g0, g1, bias):
    half = g0.shape[1] - 16
    return fn(sd2, _pad_tab(ald), g0, g1, bias.reshape(2, half))


def kernel(x, edge_index, W1, a_src1, a_dst1, b1, ln1_g, ln1_b,
           W2, a_src2, a_dst2, b2, ln2_g, ln2_b, W3, a_src3, a_dst3, b3):
    loop = jnp.arange(N, dtype=edge_index.dtype)
    pad = E2P - E2
    src = jnp.concatenate([edge_index[0], loop, jnp.zeros((pad,), edge_index.dtype)])
    dst = jnp.concatenate([edge_index[1], loop, jnp.full((pad,), TRASH, edge_index.dtype)])
    def mk_sd2(b):
        return jnp.concatenate([src.reshape(E2P // b, b), dst.reshape(E2P // b, b)], axis=1)

    B_FULL, NS_FULL = 64, 3
    B_OUT, NS_OUT = 64, 3
    sd2 = mk_sd2(B_FULL)
    sd2_out = mk_sd2(B_OUT)
    sc_full = _make_sc_layer(HC // 2, C, B_FULL, NS_FULL)
    sc_out = _make_sc_layer(NUM_CLASSES // 2, 64, B_OUT, NS_OUT)

    as1, ad1 = _attn_mats(a_src1, a_dst1, H, C)
    g0, g1, ald = _tc_first(x, W1, as1, ad1)
    o0, o1 = _sc_edge_stage(sc_full, sd2, ald, g0, g1, b1)

    as2, ad2 = _attn_mats(a_src2, a_dst2, H, C)
    y1, g0, g1, ald = _tc_mid(o0, o1, x, ln1_g, ln1_b, W2, as2, ad2)
    o0, o1 = _sc_edge_stage(sc_full, sd2, ald, g0, g1, b2)

    as3, ad3 = _attn_mats(a_src3, a_dst3, 1, NUM_CLASSES)
    _, g0, g1, ald = _tc_mid(o0, o1, y1, ln2_g, ln2_b, W3, as3, ad3)
    o0, o1 = _sc_edge_stage(sc_out, sd2_out, ald, g0, g1, b3)
    half = NUM_CLASSES // 2
    return jnp.concatenate([o0[:N, :half], o1[:N, :half]], axis=1)


# speedup vs baseline: 1.3779x; 1.3779x over previous
import functools

import jax
import jax.numpy as jnp
from jax import lax
from jax.experimental import pallas as pl
from jax.experimental.pallas import tpu as pltpu
from jax.experimental.pallas import tpu_sc as plsc

N = 10000
E = 160000
D = 256
H = 8
C = 32
HC = H * C
NUM_CLASSES = 64

NSUB = 16
NP = 10112
TRASH = N
E2 = E + N
E2P = 172032
CHUNK = E2P // NSUB
ROWS = NP // NSUB

BN = 1000


def _augment(h, als, half):
    pad = jnp.zeros((h.shape[0], 16 - H), jnp.float32)
    return (jnp.concatenate([h[:, :half], als, pad], axis=1),
            jnp.concatenate([h[:, half:], als, pad], axis=1))


def _tc_first_body(x_ref, w_ref, as_ref, ad_ref, g0_ref, g1_ref, do_ref):
    h = jnp.dot(x_ref[...], w_ref[...], preferred_element_type=jnp.float32)
    half = h.shape[1] // 2
    als = jnp.dot(h, as_ref[...], preferred_element_type=jnp.float32)
    g0_ref[...], g1_ref[...] = _augment(h, als, half)
    do_ref[...] = jnp.dot(h, ad_ref[...], preferred_element_type=jnp.float32)


def _tc_mid_body(z0_ref, z1_ref, r_ref, g_ref, b_ref, w_ref, as_ref, ad_ref,
                 y_ref, g0_ref, g1_ref, do_ref):
    halfp = z0_ref.shape[1] - 16
    z = jnp.concatenate([z0_ref[:, :halfp], z1_ref[:, :halfp]], axis=1) + r_ref[...]
    mu = jnp.mean(z, axis=-1, keepdims=True)
    var = jnp.mean((z - mu) ** 2, axis=-1, keepdims=True)
    y = (z - mu) / jnp.sqrt(var + 1e-5) * g_ref[...] + b_ref[...]
    y = jnp.where(y > 0, y, jnp.exp(jnp.minimum(y, 0.0)) - 1.0)
    y_ref[...] = y
    h = jnp.dot(y, w_ref[...], preferred_element_type=jnp.float32)
    half = h.shape[1] // 2
    als = jnp.dot(h, as_ref[...], preferred_element_type=jnp.float32)
    g0_ref[...], g1_ref[...] = _augment(h, als, half)
    do_ref[...] = jnp.dot(h, ad_ref[...], preferred_element_type=jnp.float32)


def _tc_first(x, w, a_s, a_d):
    k, m = w.shape
    wid = m // 2 + 16
    f32 = jnp.float32
    return pl.pallas_call(
        _tc_first_body,
        grid=(N // BN,),
        in_specs=[
            pl.BlockSpec((BN, k), lambda i: (i, 0)),
            pl.BlockSpec((k, m), lambda i: (0, 0)),
            pl.BlockSpec((m, H), lambda i: (0, 0)),
            pl.BlockSpec((m, H), lambda i: (0, 0)),
        ],
        out_specs=[
            pl.BlockSpec((BN, wid), lambda i: (i, 0)),
            pl.BlockSpec((BN, wid), lambda i: (i, 0)),
            pl.BlockSpec((BN, H), lambda i: (i, 0)),
        ],
        out_shape=[
            jax.ShapeDtypeStruct((N, wid), f32),
            jax.ShapeDtypeStruct((N, wid), f32),
            jax.ShapeDtypeStruct((N, H), f32),
        ],
    )(x, w, a_s, a_d)


def _tc_mid(z0, z1, r, ln_g, ln_b, w, a_s, a_d):
    k, m = w.shape
    widp = z0.shape[1]
    wid = m // 2 + 16
    f32 = jnp.float32
    return pl.pallas_call(
        _tc_mid_body,
        grid=(N // BN,),
        in_specs=[
            pl.BlockSpec((BN, widp), lambda i: (i, 0)),
            pl.BlockSpec((BN, widp), lambda i: (i, 0)),
            pl.BlockSpec((BN, k), lambda i: (i, 0)),
            pl.BlockSpec((1, k), lambda i: (0, 0)),
            pl.BlockSpec((1, k), lambda i: (0, 0)),
            pl.BlockSpec((k, m), lambda i: (0, 0)),
            pl.BlockSpec((m, H), lambda i: (0, 0)),
            pl.BlockSpec((m, H), lambda i: (0, 0)),
        ],
        out_specs=[
            pl.BlockSpec((BN, k), lambda i: (i, 0)),
            pl.BlockSpec((BN, wid), lambda i: (i, 0)),
            pl.BlockSpec((BN, wid), lambda i: (i, 0)),
            pl.BlockSpec((BN, H), lambda i: (i, 0)),
        ],
        out_shape=[
            jax.ShapeDtypeStruct((N, k), f32),
            jax.ShapeDtypeStruct((N, wid), f32),
            jax.ShapeDtypeStruct((N, wid), f32),
            jax.ShapeDtypeStruct((N, H), f32),
        ],
    )(z0, z1, r, ln_g.reshape(1, k), ln_b.reshape(1, k), w, a_s, a_d)


def _splat(v, idx):
    dnums = lax.GatherDimensionNumbers(
        offset_dims=(), collapsed_slice_dims=(0,), start_index_map=(0,))
    return lax.gather(v, idx[:, None], dnums, (1,),
                      mode=lax.GatherScatterMode.PROMISE_IN_BOUNDS)


@functools.cache
def _make_sc_layer(half, c_l, B, NS, interpret=False):
    nj = half // 16
    f32 = jnp.float32
    mesh = plsc.VectorSubcoreMesh(core_axis_name="c", subcore_axis_name="s",
                                  num_cores=2, num_subcores=NSUB)

    def _head_splat(p, c, j):
        if c_l == 32:
            idx = jnp.full((16,), j // 2, jnp.int32) + c * (half // 32)
        else:
            idx = jnp.zeros((16,), jnp.int32)
        return _splat(p, idx)

    WID = half + 16
    TAIL = half
    T = CHUNK // B

    def body(*refs):
        (sd2_h, adt_h, h0_h, h1_h, bias_h, o0_h, o1_h,
         acc, sdbuf, dsc, adb, g, bias_v) = refs[:13]
        sems = refs[13:]
        ISEM = sems[0:NS]
        BSEM = sems[NS:2 * NS]
        GSEM = sems[2 * NS:3 * NS]
        SSEM = sems[3 * NS:4 * NS]
        c = lax.axis_index("c")
        s = lax.axis_index("s")
        rbase = s * ROWS

        def idx_issue(t, k):
            row = s * T + t
            pltpu.async_copy(sd2_h.at[pl.ds(row, 1)], sdbuf.at[pl.ds(k, 1)], ISEM[k])

        def idx_wait(k):
            pltpu.make_async_copy(sd2_h.at[pl.ds(0, 1)], sdbuf.at[pl.ds(k, 1)], ISEM[k]).wait()

        def gathers_issue(k):
            def sh(q, _):
                dsc[k, pl.ds(16 * q, 16)] = sdbuf[k, pl.ds(B + 16 * q, 16)]
                return 0
            lax.fori_loop(0, B // 16, sh, 0, unroll=True)
            pltpu.async_copy(adt_h.at[dsc.at[k]], adb.at[k], BSEM[k])

            @pl.when(c == 0)
            def _():
                pltpu.async_copy(h0_h.at[sdbuf.at[k, pl.ds(0, B)]], g.at[k], GSEM[k])

            @pl.when(c == 1)
            def _():
                pltpu.async_copy(h1_h.at[sdbuf.at[k, pl.ds(0, B)]], g.at[k], GSEM[k])

        def gathers_wait(k):
            pltpu.make_async_copy(adt_h.at[dsc.at[k]], adb.at[k], BSEM[k]).wait()
            pltpu.make_async_copy(h0_h.at[sdbuf.at[k, pl.ds(0, B)]], g.at[k], GSEM[k]).wait()

        def scatter_issue(k):
            pltpu.async_copy(g.at[k], acc.at[dsc.at[k]], SSEM[k], add=True)

        def scatter_drain(k):
            pltpu.make_async_copy(g.at[k], acc.at[dsc.at[k]], SSEM[k]).wait()

        pltpu.sync_copy(bias_h.at[pl.ds(c, 1)], bias_v)

        def _fill(i, _):
            for j in range(WID // 16):
                g[0, i, pl.ds(16 * j, 16)] = jnp.zeros((16,), f32)
            return 0
        lax.fori_loop(0, B, _fill, 0)
        zchunks = [(q * B, B) for q in range(ROWS // B)]
        if ROWS % B:
            zchunks.append((ROWS - ROWS % B, ROWS % B))
        for off, sz in zchunks:
            pltpu.async_copy(g.at[0, pl.ds(0, sz)], acc.at[pl.ds(rbase + off, sz)], GSEM[0])
        for off, sz in zchunks:
            pltpu.make_async_copy(g.at[0, pl.ds(0, sz)], acc.at[pl.ds(rbase + off, sz)], GSEM[0]).wait()
        plsc.subcore_barrier()

        def compute(k):
            def me(i, _):
                e = g[k, i, pl.ds(TAIL, 16)] + adb[k, i]
                e = jnp.maximum(e, 0.2 * e)
                p = jnp.exp(e)
                g[k, i, pl.ds(TAIL, 16)] = p
                step = max(1, 32 // c_l)
                sps = {j: _head_splat(p, c, j) for j in range(0, nj, step)}
                for j in range(nj):
                    g[k, i, pl.ds(16 * j, 16)] = g[k, i, pl.ds(16 * j, 16)] * sps[(j // step) * step]
                return 0
            lax.fori_loop(0, B, me, 0, unroll=2)

        idx_issue(0, 0)
        idx_issue(1, 1)
        idx_wait(0)
        gathers_issue(0)

        def tstep(t3, _):
            for ks in range(NS):
                t = NS * t3 + ks
                k1 = (ks + 1) % NS

                if NS > 2:
                    @pl.when(t + 2 < T)
                    def _():
                        idx_issue(t + 2, (ks + 2) % NS)

                @pl.when(t + 1 < T)
                def _():
                    idx_wait(k1)

                    @pl.when(t + 1 >= NS)
                    def _():
                        scatter_drain(k1)
                    gathers_issue(k1)
                gathers_wait(ks)
                if NS == 2:
                    @pl.when(t + 2 < T)
                    def _():
                        idx_issue(t + 2, ks)
                compute(ks)
                scatter_issue(ks)
            return 0
        lax.fori_loop(0, T // NS, tstep, 0)
        for ks in range(NS):
            scatter_drain(ks)
        plsc.subcore_barrier()

        rounds = []
        off = 0
        while off < ROWS:
            sizes = []
            for k in range(NS):
                sz = min(B, ROWS - off - sum(sizes))
                if sz > 0:
                    sizes.append(sz)
            rounds.append((off, sizes))
            off += sum(sizes)

        for off, sizes in rounds:
            for k, sz in enumerate(sizes):
                r0 = rbase + off + k * B
                pltpu.async_copy(acc.at[pl.ds(r0, sz)], g.at[k, pl.ds(0, sz)], GSEM[k])
            for k, sz in enumerate(sizes):
                r0 = rbase + off + k * B
                pltpu.make_async_copy(acc.at[pl.ds(r0, sz)], g.at[k, pl.ds(0, sz)], GSEM[k]).wait()

                def ne(i, _, k=k):
                    d = g[k, i, pl.ds(TAIL, 16)] + 1e-16
                    for j in range(nj):
                        bv = bias_v[0, pl.ds(16 * j, 16)]
                        g[k, i, pl.ds(16 * j, 16)] = g[k, i, pl.ds(16 * j, 16)] / _head_splat(d, c, j) + bv
                    return 0
                lax.fori_loop(0, sz, ne, 0, unroll=2)

                @pl.when(c == 0)
                def _():
                    pltpu.async_copy(g.at[k, pl.ds(0, sz)], o0_h.at[pl.ds(r0, sz)], BSEM[k])

                @pl.when(c == 1)
                def _():
                    pltpu.async_copy(g.at[k, pl.ds(0, sz)], o1_h.at[pl.ds(r0, sz)], BSEM[k])
            for k, sz in enumerate(sizes):
                r0 = rbase + off + k * B

                @pl.when(c == 0)
                def _():
                    pltpu.make_async_copy(g.at[k, pl.ds(0, sz)], o0_h.at[pl.ds(r0, sz)], BSEM[k]).wait()

                @pl.when(c == 1)
                def _():
                    pltpu.make_async_copy(g.at[k, pl.ds(0, sz)], o1_h.at[pl.ds(r0, sz)], BSEM[k]).wait()

    return pl.kernel(
        body,
        out_type=(
            jax.ShapeDtypeStruct((NP, WID), f32),
            jax.ShapeDtypeStruct((NP, WID), f32),
        ),
        mesh=mesh,
        interpret=interpret,
        compiler_params=pltpu.CompilerParams(use_tc_tiling_on_sc=False),
        scratch_types=[
            pltpu.VMEM_SHARED((NP, WID), f32),
            pltpu.VMEM((NS, 2 * B), jnp.int32),
            pltpu.VMEM((NS, B), jnp.int32),
            pltpu.VMEM((NS, B, 16), f32),
            pltpu.VMEM((NS, B, WID), f32),
            pltpu.VMEM((1, half), f32),
        ] + [pltpu.SemaphoreType.DMA] * (4 * NS),
    )


def _attn_mats(a_src, a_dst, heads, c_out):
    eye = jnp.eye(heads, dtype=jnp.float32)
    a_s = (a_src[:, :, None] * eye[:, None, :]).reshape(heads * c_out, heads)
    a_d = (a_dst[:, :, None] * eye[:, None, :]).reshape(heads * c_out, heads)
    if heads < H:
        a_s = jnp.pad(a_s, ((0, 0), (0, H - heads)))
        a_d = jnp.pad(a_d, ((0, 0), (0, H - heads)))
    return a_s, a_d


def _pad_tab(a):
    return jnp.pad(a, ((0, NP - N), (0, 16 - H)))


def _sc_edge_stage(fn, sd2, ald, g0, g1, bias):
    half = g0.shape[1] - 16
    return fn(sd2, _pad_tab(ald), g0, g1, bias.reshape(2, half))


def kernel(x, edge_index, W1, a_src1, a_dst1, b1, ln1_g, ln1_b,
           W2, a_src2, a_dst2, b2, ln2_g, ln2_b, W3, a_src3, a_dst3, b3):
    loop = jnp.arange(N, dtype=edge_index.dtype)
    pad = E2P - E2
    src = jnp.concatenate([edge_index[0], loop, jnp.zeros((pad,), edge_index.dtype)])
    dst = jnp.concatenate([edge_index[1], loop, jnp.full((pad,), TRASH, edge_index.dtype)])
    def mk_sd2(b):
        return jnp.concatenate([src.reshape(E2P // b, b), dst.reshape(E2P // b, b)], axis=1)

    B_FULL, NS_FULL = 64, 3
    B_OUT, NS_OUT = 64, 3
    sd2 = mk_sd2(B_FULL)
    sd2_out = mk_sd2(B_OUT)
    sc_full = _make_sc_layer(HC // 2, C, B_FULL, NS_FULL)
    sc_out = _make_sc_layer(NUM_CLASSES // 2, 64, B_OUT, NS_OUT)

    as1, ad1 = _attn_mats(a_src1, a_dst1, H, C)
    g0, g1, ald = _tc_first(x, W1, as1, ad1)
    o0, o1 = _sc_edge_stage(sc_full, sd2, ald, g0, g1, b1)

    as2, ad2 = _attn_mats(a_src2, a_dst2, H, C)
    y1, g0, g1, ald = _tc_mid(o0, o1, x, ln1_g, ln1_b, W2, as2, ad2)
    o0, o1 = _sc_edge_stage(sc_full, sd2, ald, g0, g1, b2)

    as3, ad3 = _attn_mats(a_src3, a_dst3, 1, NUM_CLASSES)
    _, g0, g1, ald = _tc_mid(o0, o1, y1, ln2_g, ln2_b, W3, as3, ad3)
    o0, o1 = _sc_edge_stage(sc_out, sd2_out, ald, g0, g1, b3)
    half = NUM_CLASSES // 2
    return jnp.concatenate([o0[:N, :half], o1[:N, :half]], axis=1)
